# Initial kernel scaffold; baseline (speedup 1.0000x reference)
import jax
import jax.numpy as jnp
from jax.experimental import pallas as pl


def kernel(x, x_clique, node2clique_index, W, b):
    raise NotImplementedError("write your pallas kernel here")


# speedup vs baseline: 4.0517x; 4.0517x over previous
import jax
import jax.numpy as jnp
from jax import lax
from jax.experimental import pallas as pl
from jax.experimental.pallas import tpu as pltpu
from jax.experimental.pallas import tpu_sc as plsc

N_NODES = 10000
N_CLIQUES = 5000
N_EDGES = 320000
D = 128

NC = 2
NS = 16
NW = NC * NS
L = 16
CH = 128
NCHUNK = 80
E_PER = NCHUNK * CH
E_PAD = NW * E_PER
TRASH = 5119
C_PAD = 5120
ROWS_PER_TILE = C_PAD // NS
ZR = 32


def _sc_body(x_hbm, nidx_hbm, cidx_hbm, zeros_hbm,
             psum_hbm, pcnt_hbm,
             nrow, crow, buf0, buf1, cnt_v, zv,
             acc, sem0, sem1):
    cid = lax.axis_index("c")
    sid = lax.axis_index("s")
    wid = cid * NS + sid

    pltpu.sync_copy(zeros_hbm, zv)

    z16 = jnp.zeros((L,), jnp.float32)

    @pl.loop(0, C_PAD // L)
    def _zero_cnt(i):
        cnt_v[pl.ds(i * L, L)] = z16

    r0 = sid * ROWS_PER_TILE
    for z in range(ROWS_PER_TILE // ZR):
        pltpu.sync_copy(zv, acc.at[pl.ds(r0 + z * ZR, ZR)])
    plsc.subcore_barrier()

    one16 = jnp.ones((L,), jnp.float32)

    def loop_body(j, _):
        pltpu.sync_copy(nidx_hbm.at[wid, j], nrow)
        pltpu.sync_copy(cidx_hbm.at[wid, j], crow)
        pltpu.async_copy(x_hbm.at[nrow], buf0, sem0).wait()
        pltpu.sync_copy(buf0, acc.at[crow], add=True)
        for k in range(CH // L):
            idx = crow[pl.ds(k * L, L)]
            plsc.addupdate_scatter(cnt_v, [idx], one16)
        return 0

    lax.fori_loop(0, NCHUNK, loop_body, 0)

    plsc.subcore_barrier()

    pltpu.sync_copy(acc.at[pl.ds(r0, ROWS_PER_TILE)],
                    psum_hbm.at[cid, pl.ds(r0, ROWS_PER_TILE)])
    pltpu.sync_copy(cnt_v, pcnt_hbm.at[wid])


@jax.jit
def _sc_segment_sum(x, nidx, cidx, zeros):
    mesh = plsc.VectorSubcoreMesh(core_axis_name="c", subcore_axis_name="s",
                                  num_cores=NC, num_subcores=NS)
    return pl.kernel(
        _sc_body,
        out_type=[
            jax.ShapeDtypeStruct((NC, C_PAD, D), jnp.float32),
            jax.ShapeDtypeStruct((NW, C_PAD), jnp.float32),
        ],
        mesh=mesh,
        compiler_params=pltpu.CompilerParams(needs_layout_passes=False),
        scratch_types=[
            pltpu.VMEM((CH,), jnp.int32),
            pltpu.VMEM((CH,), jnp.int32),
            pltpu.VMEM((CH, D), jnp.float32),
            pltpu.VMEM((CH, D), jnp.float32),
            pltpu.VMEM((C_PAD,), jnp.float32),
            pltpu.VMEM((ZR, D), jnp.float32),
            pltpu.VMEM_SHARED((C_PAD, D), jnp.float32),
            pltpu.SemaphoreType.DMA,
            pltpu.SemaphoreType.DMA,
        ],
    )(x, nidx, cidx, zeros)


def _tc_body(psum_ref, pcnt_ref, w_ref, b_ref, out_ref):
    s = psum_ref[0] + psum_ref[1]
    c = jnp.sum(pcnt_ref[...], axis=1, keepdims=True)
    mean = s / jnp.maximum(c, 1.0)
    out_ref[...] = lax.dot_general(
        mean, w_ref[...], (((1,), (1,)), ((), ())),
        preferred_element_type=jnp.float32) + b_ref[...]


@jax.jit
def _tc_finish(psum, pcnt_t, W, b2d):
    return pl.pallas_call(
        _tc_body,
        out_shape=jax.ShapeDtypeStruct((C_PAD, D), jnp.float32),
    )(psum, pcnt_t, W, b2d)


def kernel(x, x_clique, node2clique_index, W, b):
    pad = E_PAD - N_EDGES
    nidx = jnp.concatenate(
        [node2clique_index[0], jnp.zeros((pad,), jnp.int32)]
    ).reshape(NW, NCHUNK, CH)
    cidx = jnp.concatenate(
        [node2clique_index[1], jnp.full((pad,), TRASH, jnp.int32)]
    ).reshape(NW, NCHUNK, CH)
    zeros = jnp.zeros((ZR, D), jnp.float32)
    psum, pcnt = _sc_segment_sum(x, nidx, cidx, zeros)
    out = _tc_finish(psum, pcnt.T, W, b.reshape(1, D))
    return out[:N_CLIQUES]


# speedup vs baseline: 5.1340x; 1.2671x over previous
import jax
import jax.numpy as jnp
from jax import lax
from jax.experimental import pallas as pl
from jax.experimental.pallas import tpu as pltpu
from jax.experimental.pallas import tpu_sc as plsc

N_NODES = 10000
N_CLIQUES = 5000
N_EDGES = 320000
D = 128

NC = 2
NS = 16
NW = NC * NS
L = 16
CH = 128
NCHUNK = 80
E_PER = NCHUNK * CH
E_PAD = NW * E_PER
TRASH = 5119
C_PAD = 5120
ROWS_PER_TILE = C_PAD // NS
ZR = 32


def _sc_body(x_hbm, nidx_hbm, cidx_hbm, zeros_hbm,
             psum_hbm, pcnt_hbm,
             nidx_all, cidx_all, nrow0, crow0, nrow1, crow1,
             buf0, buf1, cnt_v, zv,
             acc, sem_g0, sem_g1, sem_s0, sem_s1):
    cid = lax.axis_index("c")
    sid = lax.axis_index("s")
    wid = cid * NS + sid

    pltpu.sync_copy(nidx_hbm.at[wid], nidx_all)
    pltpu.sync_copy(cidx_hbm.at[wid], cidx_all)
    pltpu.sync_copy(zeros_hbm, zv)

    z16 = jnp.zeros((L,), jnp.float32)

    @pl.loop(0, C_PAD // L)
    def _zero_cnt(i):
        cnt_v[pl.ds(i * L, L)] = z16

    r0 = sid * ROWS_PER_TILE
    for z in range(ROWS_PER_TILE // ZR):
        pltpu.sync_copy(zv, acc.at[pl.ds(r0 + z * ZR, ZR)])
    plsc.subcore_barrier()

    one16 = jnp.ones((L,), jnp.float32)

    def load_rows(j, nr, cr):
        for k in range(CH // L):
            nr[pl.ds(k * L, L)] = nidx_all[j, pl.ds(k * L, L)]
            cr[pl.ds(k * L, L)] = cidx_all[j, pl.ds(k * L, L)]

    def counts(j):
        for k in range(CH // L):
            idx = cidx_all[j, pl.ds(k * L, L)]
            plsc.addupdate_scatter(cnt_v, [idx], one16)

    def gather_start(nr, buf, sem):
        pltpu.async_copy(x_hbm.at[nr], buf, sem)

    def gather_wait(nr, buf, sem):
        pltpu.make_async_copy(x_hbm.at[nr], buf, sem).wait()

    def scatter_start(buf, cr, sem):
        pltpu.async_copy(buf, acc.at[cr], sem, add=True)

    def scatter_wait(buf, cr, sem):
        pltpu.make_async_copy(buf, acc.at[cr], sem).wait()

    load_rows(0, nrow0, crow0)
    gather_start(nrow0, buf0, sem_g0)
    load_rows(1, nrow1, crow1)
    gather_start(nrow1, buf1, sem_g1)

    def loop_body(jj, _):
        j0 = 2 * jj
        j1 = j0 + 1
        gather_wait(nrow0, buf0, sem_g0)
        scatter_start(buf0, crow0, sem_s0)
        counts(j0)
        gather_wait(nrow1, buf1, sem_g1)
        scatter_start(buf1, crow1, sem_s1)
        counts(j1)
        scatter_wait(buf0, crow0, sem_s0)
        load_rows(j0 + 2, nrow0, crow0)
        gather_start(nrow0, buf0, sem_g0)
        scatter_wait(buf1, crow1, sem_s1)
        load_rows(j1 + 2, nrow1, crow1)
        gather_start(nrow1, buf1, sem_g1)
        return 0

    lax.fori_loop(0, NCHUNK // 2 - 1, loop_body, 0)

    gather_wait(nrow0, buf0, sem_g0)
    scatter_start(buf0, crow0, sem_s0)
    counts(NCHUNK - 2)
    gather_wait(nrow1, buf1, sem_g1)
    scatter_start(buf1, crow1, sem_s1)
    counts(NCHUNK - 1)
    scatter_wait(buf0, crow0, sem_s0)
    scatter_wait(buf1, crow1, sem_s1)

    plsc.subcore_barrier()

    pltpu.sync_copy(acc.at[pl.ds(r0, ROWS_PER_TILE)],
                    psum_hbm.at[cid, pl.ds(r0, ROWS_PER_TILE)])
    pltpu.sync_copy(cnt_v, pcnt_hbm.at[wid])


@jax.jit
def _sc_segment_sum(x, nidx, cidx, zeros):
    mesh = plsc.VectorSubcoreMesh(core_axis_name="c", subcore_axis_name="s",
                                  num_cores=NC, num_subcores=NS)
    return pl.kernel(
        _sc_body,
        out_type=[
            jax.ShapeDtypeStruct((NC, C_PAD, D), jnp.float32),
            jax.ShapeDtypeStruct((NW, C_PAD), jnp.float32),
        ],
        mesh=mesh,
        compiler_params=pltpu.CompilerParams(needs_layout_passes=False),
        scratch_types=[
            pltpu.VMEM((NCHUNK, CH), jnp.int32),
            pltpu.VMEM((NCHUNK, CH), jnp.int32),
            pltpu.VMEM((CH,), jnp.int32),
            pltpu.VMEM((CH,), jnp.int32),
            pltpu.VMEM((CH,), jnp.int32),
            pltpu.VMEM((CH,), jnp.int32),
            pltpu.VMEM((CH, D), jnp.float32),
            pltpu.VMEM((CH, D), jnp.float32),
            pltpu.VMEM((C_PAD,), jnp.float32),
            pltpu.VMEM((ZR, D), jnp.float32),
            pltpu.VMEM_SHARED((C_PAD, D), jnp.float32),
            pltpu.SemaphoreType.DMA,
            pltpu.SemaphoreType.DMA,
            pltpu.SemaphoreType.DMA,
            pltpu.SemaphoreType.DMA,
        ],
    )(x, nidx, cidx, zeros)


def _tc_body(psum_ref, pcnt_ref, w_ref, b_ref, out_ref):
    s = psum_ref[0] + psum_ref[1]
    c = jnp.sum(pcnt_ref[...], axis=1, keepdims=True)
    mean = s / jnp.maximum(c, 1.0)
    out_ref[...] = lax.dot_general(
        mean, w_ref[...], (((1,), (1,)), ((), ())),
        preferred_element_type=jnp.float32) + b_ref[...]


@jax.jit
def _tc_finish(psum, pcnt_t, W, b2d):
    return pl.pallas_call(
        _tc_body,
        out_shape=jax.ShapeDtypeStruct((C_PAD, D), jnp.float32),
    )(psum, pcnt_t, W, b2d)


def kernel(x, x_clique, node2clique_index, W, b):
    pad = E_PAD - N_EDGES
    nidx = jnp.concatenate(
        [node2clique_index[0], jnp.zeros((pad,), jnp.int32)]
    ).reshape(NW, NCHUNK, CH)
    cidx = jnp.concatenate(
        [node2clique_index[1], jnp.full((pad,), TRASH, jnp.int32)]
    ).reshape(NW, NCHUNK, CH)
    zeros = jnp.zeros((ZR, D), jnp.float32)
    psum, pcnt = _sc_segment_sum(x, nidx, cidx, zeros)
    out = _tc_finish(psum, pcnt.T, W, b.reshape(1, D))
    return out[:N_CLIQUES]


# speedup vs baseline: 5.2236x; 1.0174x over previous
import jax
import jax.numpy as jnp
from jax import lax
from jax.experimental import pallas as pl
from jax.experimental.pallas import tpu as pltpu
from jax.experimental.pallas import tpu_sc as plsc

N_NODES = 10000
N_CLIQUES = 5000
N_EDGES = 320000
D = 128

NC = 2
NS = 16
NW = NC * NS
L = 16
CH = 64
NCHUNK = 160
NSLOT = 4
NGROUP = NCHUNK // NSLOT
E_PER = NCHUNK * CH
E_PAD = NW * E_PER
TRASH = 5119
C_PAD = 5120
ROWS_PER_TILE = C_PAD // NS
ZR = 32


def _sc_body(x_hbm, nidx_hbm, cidx_hbm, zeros_hbm,
             psum_hbm, pcnt_hbm,
             nidx_all, cidx_all,
             nrow0, crow0, nrow1, crow1, nrow2, crow2, nrow3, crow3,
             buf0, buf1, buf2, buf3, cnt_v, zv,
             acc, sg0, sg1, sg2, sg3, ss0, ss1, ss2, ss3):
    nrows = [nrow0, nrow1, nrow2, nrow3]
    crows = [crow0, crow1, crow2, crow3]
    bufs = [buf0, buf1, buf2, buf3]
    sgs = [sg0, sg1, sg2, sg3]
    sss = [ss0, ss1, ss2, ss3]

    cid = lax.axis_index("c")
    sid = lax.axis_index("s")
    wid = cid * NS + sid

    pltpu.sync_copy(nidx_hbm.at[wid], nidx_all)
    pltpu.sync_copy(cidx_hbm.at[wid], cidx_all)
    pltpu.sync_copy(zeros_hbm, zv)

    z16 = jnp.zeros((L,), jnp.float32)

    @pl.loop(0, C_PAD // L)
    def _zero_cnt(i):
        cnt_v[pl.ds(i * L, L)] = z16

    r0 = sid * ROWS_PER_TILE
    for z in range(ROWS_PER_TILE // ZR):
        pltpu.sync_copy(zv, acc.at[pl.ds(r0 + z * ZR, ZR)])
    plsc.subcore_barrier()

    one16 = jnp.ones((L,), jnp.float32)

    def load_rows(j, nr, cr):
        for k in range(CH // L):
            nr[pl.ds(k * L, L)] = nidx_all[j, pl.ds(k * L, L)]
            cr[pl.ds(k * L, L)] = cidx_all[j, pl.ds(k * L, L)]

    def counts(j):
        for k in range(CH // L):
            idx = cidx_all[j, pl.ds(k * L, L)]
            plsc.addupdate_scatter(cnt_v, [idx], one16)

    def gather_start(s):
        pltpu.async_copy(x_hbm.at[nrows[s]], bufs[s], sgs[s])

    def gather_wait(s):
        pltpu.make_async_copy(x_hbm.at[nrows[s]], bufs[s], sgs[s]).wait()

    def scatter_start(s):
        pltpu.async_copy(bufs[s], acc.at[crows[s]], sss[s], add=True)

    def scatter_wait(s):
        pltpu.make_async_copy(bufs[s], acc.at[crows[s]], sss[s]).wait()

    for s in range(NSLOT):
        load_rows(s, nrows[s], crows[s])
        gather_start(s)

    def loop_body(g, _):
        j = g * NSLOT
        for s in range(NSLOT):
            gather_wait(s)
            scatter_start(s)
            counts(j + s)
        for s in range(NSLOT):
            scatter_wait(s)
            load_rows(j + NSLOT + s, nrows[s], crows[s])
            gather_start(s)
        return 0

    lax.fori_loop(0, NGROUP - 1, loop_body, 0)

    jlast = (NGROUP - 1) * NSLOT
    for s in range(NSLOT):
        gather_wait(s)
        scatter_start(s)
        counts(jlast + s)
    for s in range(NSLOT):
        scatter_wait(s)

    plsc.subcore_barrier()

    pltpu.sync_copy(acc.at[pl.ds(r0, ROWS_PER_TILE)],
                    psum_hbm.at[cid, pl.ds(r0, ROWS_PER_TILE)])
    pltpu.sync_copy(cnt_v, pcnt_hbm.at[wid])


@jax.jit
def _sc_segment_sum(x, nidx, cidx, zeros):
    mesh = plsc.VectorSubcoreMesh(core_axis_name="c", subcore_axis_name="s",
                                  num_cores=NC, num_subcores=NS)
    return pl.kernel(
        _sc_body,
        out_type=[
            jax.ShapeDtypeStruct((NC, C_PAD, D), jnp.float32),
            jax.ShapeDtypeStruct((NW, C_PAD), jnp.float32),
        ],
        mesh=mesh,
        compiler_params=pltpu.CompilerParams(needs_layout_passes=False),
        scratch_types=[
            pltpu.VMEM((NCHUNK, CH), jnp.int32),
            pltpu.VMEM((NCHUNK, CH), jnp.int32),
        ] + [pltpu.VMEM((CH,), jnp.int32) for _ in range(2 * NSLOT)] + [
            pltpu.VMEM((CH, D), jnp.float32) for _ in range(NSLOT)] + [
            pltpu.VMEM((C_PAD,), jnp.float32),
            pltpu.VMEM((ZR, D), jnp.float32),
            pltpu.VMEM_SHARED((C_PAD, D), jnp.float32),
        ] + [pltpu.SemaphoreType.DMA for _ in range(2 * NSLOT)],
    )(x, nidx, cidx, zeros)


def _tc_body(psum_ref, pcnt_ref, w_ref, b_ref, out_ref):
    s = psum_ref[0] + psum_ref[1]
    c = jnp.sum(pcnt_ref[...], axis=1, keepdims=True)
    mean = s / jnp.maximum(c, 1.0)
    out_ref[...] = lax.dot_general(
        mean, w_ref[...], (((1,), (1,)), ((), ())),
        preferred_element_type=jnp.float32) + b_ref[...]


@jax.jit
def _tc_finish(psum, pcnt_t, W, b2d):
    return pl.pallas_call(
        _tc_body,
        out_shape=jax.ShapeDtypeStruct((C_PAD, D), jnp.float32),
    )(psum, pcnt_t, W, b2d)


def kernel(x, x_clique, node2clique_index, W, b):
    pad = E_PAD - N_EDGES
    nidx = jnp.concatenate(
        [node2clique_index[0], jnp.zeros((pad,), jnp.int32)]
    ).reshape(NW, NCHUNK, CH)
    cidx = jnp.concatenate(
        [node2clique_index[1], jnp.full((pad,), TRASH, jnp.int32)]
    ).reshape(NW, NCHUNK, CH)
    zeros = jnp.zeros((ZR, D), jnp.float32)
    psum, pcnt = _sc_segment_sum(x, nidx, cidx, zeros)
    out = _tc_finish(psum, pcnt.T, W, b.reshape(1, D))
    return out[:N_CLIQUES]
